# baseline (speedup 1.0000x reference)
import functools

import jax
import jax.numpy as jnp
from jax import lax
from jax.experimental import pallas as pl
from jax.experimental.pallas import tpu as pltpu
from jax.experimental.pallas import tpu_sc as plsc

BATCH = 16384
EMBED_DIM = 32
NUM_ROWS = 1000000
PACK = 4
PAD_DIM = PACK * EMBED_DIM
PACKED_ROWS = 253952
NUM_CORES = 2
NUM_SUBCORES = 16
NUM_WORKERS = NUM_CORES * NUM_SUBCORES
ROWS_PER_WORKER = BATCH // NUM_WORKERS
CHUNK = 128
NUM_CHUNKS = ROWS_PER_WORKER // CHUNK
LANES = 16


RELAY_P = 4096
RELAY_GRID = PACKED_ROWS // RELAY_P


def _relayout_body(u0, u1, u2, u3, i0, i1, i2, i3, ou_ref, oi_ref):
    eye = jnp.eye(EMBED_DIM, dtype=jnp.float32)
    tr = lambda x: lax.dot_general(
        x[...], eye, (((0,), (0,)), ((), ())),
        preferred_element_type=jnp.float32)
    ou_ref[...] = jnp.concatenate([tr(x) for x in (u0, u1, u2, u3)], axis=1)
    oi_ref[...] = jnp.concatenate([tr(x) for x in (i0, i1, i2, i3)], axis=1)


def _relayout(utabT, itabT):
    spec = [
        pl.BlockSpec(
            (EMBED_DIM, RELAY_P),
            functools.partial(
                lambda k, i: (0, jnp.minimum(i + k * RELAY_GRID,
                                             NUM_ROWS // RELAY_P)), k))
        for k in range(PACK)
    ]
    return pl.pallas_call(
        _relayout_body,
        grid=(RELAY_GRID,),
        in_specs=spec + spec,
        out_specs=[pl.BlockSpec((RELAY_P, PAD_DIM), lambda i: (i, 0))] * 2,
        out_shape=[jax.ShapeDtypeStruct((PACKED_ROWS, PAD_DIM), jnp.float32)] * 2,
    )(utabT, utabT, utabT, utabT, itabT, itabT, itabT, itabT)


_sc_mesh = plsc.VectorSubcoreMesh(core_axis_name="c", subcore_axis_name="s")


@functools.partial(
    pl.kernel,
    mesh=_sc_mesh,
    out_type=[
        jax.ShapeDtypeStruct((BATCH, PAD_DIM), jnp.float32),
        jax.ShapeDtypeStruct((BATCH, PAD_DIM), jnp.float32),
    ],
    scratch_types=[
        pltpu.VMEM((ROWS_PER_WORKER,), jnp.int32),
        pltpu.VMEM((ROWS_PER_WORKER,), jnp.int32),
        pltpu.VMEM((2, CHUNK, PAD_DIM), jnp.float32),
        pltpu.VMEM((2, CHUNK, PAD_DIM), jnp.float32),
        pltpu.SemaphoreType.DMA,
        pltpu.SemaphoreType.DMA,
    ],
    compiler_params=pltpu.CompilerParams(use_tc_tiling_on_sc=False),
)
def _sc_gather(uids_hbm, iids_hbm, utab_hbm, itab_hbm, uout_hbm, iout_hbm,
               uidx_v, iidx_v, upad_v, ipad_v, sem_g, sem_o):
    wid = lax.axis_index("s") * NUM_CORES + lax.axis_index("c")
    base = wid * ROWS_PER_WORKER
    pltpu.sync_copy(uids_hbm.at[wid], uidx_v)
    pltpu.sync_copy(iids_hbm.at[wid], iidx_v)
    for g in range(ROWS_PER_WORKER // LANES):
        sl = pl.ds(g * LANES, LANES)
        uidx_v[sl] = lax.rem(uidx_v[sl], PACKED_ROWS)
        iidx_v[sl] = lax.rem(iidx_v[sl], PACKED_ROWS)

    def fire(j):
        buf = j % 2
        return (
            pltpu.async_copy(
                utab_hbm.at[uidx_v.at[pl.ds(j * CHUNK, CHUNK)]],
                upad_v.at[buf], sem_g),
            pltpu.async_copy(
                itab_hbm.at[iidx_v.at[pl.ds(j * CHUNK, CHUNK)]],
                ipad_v.at[buf], sem_g),
        )

    def flush(j):
        buf = j % 2
        dst = pl.ds(base + j * CHUNK, CHUNK)
        return (
            pltpu.async_copy(upad_v.at[buf], uout_hbm.at[dst], sem_o),
            pltpu.async_copy(ipad_v.at[buf], iout_hbm.at[dst], sem_o),
        )

    gathers = fire(0)
    outs = []
    for j in range(NUM_CHUNKS):
        for c in gathers:
            c.wait()
        outs.append(flush(j))
        if j + 1 < NUM_CHUNKS:
            if j >= 1:
                for c in outs[j - 1]:
                    c.wait()
            gathers = fire(j + 1)
    for pair in outs[-2:]:
        for c in pair:
            c.wait()


MLP_BLOCK = 2048


def _mlp_body(u_ref, i_ref, uid_ref, iid_ref, w1u_ref, w1i_ref, b1_ref,
              w2_ref, b2_ref, w3_ref, b3_ref, o_ref):
    lane_group = lax.broadcasted_iota(jnp.int32, (MLP_BLOCK, PAD_DIM), 1) // EMBED_DIM
    u_sel = jnp.where(lane_group == uid_ref[...] // PACKED_ROWS, u_ref[...], 0.0)
    i_sel = jnp.where(lane_group == iid_ref[...] // PACKED_ROWS, i_ref[...], 0.0)
    h = jnp.dot(u_sel, w1u_ref[...], preferred_element_type=jnp.float32)
    h = h + jnp.dot(i_sel, w1i_ref[...], preferred_element_type=jnp.float32)
    h = jnp.maximum(h + b1_ref[...], 0.0)
    h = jnp.dot(h, w2_ref[...], preferred_element_type=jnp.float32) + b2_ref[...]
    h = jnp.maximum(h, 0.0)
    o_ref[...] = (
        jnp.dot(h, w3_ref[...], preferred_element_type=jnp.float32) + b3_ref[...]
    )


def _mlp(u_pad, i_pad, uids, iids, W1, b1, W2, b2, W3, b3):
    w1u = jnp.concatenate([W1[:EMBED_DIM]] * PACK, axis=0)
    w1i = jnp.concatenate([W1[EMBED_DIM:]] * PACK, axis=0)
    grid = (BATCH // MLP_BLOCK,)
    full = lambda shape: pl.BlockSpec(shape, lambda i: (0, 0))
    out = pl.pallas_call(
        _mlp_body,
        grid=grid,
        in_specs=[
            pl.BlockSpec((MLP_BLOCK, PAD_DIM), lambda i: (i, 0)),
            pl.BlockSpec((MLP_BLOCK, PAD_DIM), lambda i: (i, 0)),
            pl.BlockSpec((MLP_BLOCK, 1), lambda i: (i, 0)),
            pl.BlockSpec((MLP_BLOCK, 1), lambda i: (i, 0)),
            full((PAD_DIM, 64)),
            full((PAD_DIM, 64)),
            full((1, 64)),
            full((64, 32)),
            full((1, 32)),
            full((32, 1)),
            full((1, 1)),
        ],
        out_specs=pl.BlockSpec((MLP_BLOCK, 1), lambda i: (i, 0)),
        out_shape=jax.ShapeDtypeStruct((BATCH, 1), jnp.float32),
    )(u_pad, i_pad, uids.reshape(BATCH, 1), iids.reshape(BATCH, 1), w1u, w1i,
      b1.reshape(1, 64), W2, b2.reshape(1, 32), W3, b3.reshape(1, 1))
    return out[:, 0]


def kernel(user_ids, item_ids, user_table, item_table, W1, b1, W2, b2, W3, b3):
    uids = user_ids.astype(jnp.int32)
    iids = item_ids.astype(jnp.int32)
    utab, itab = _relayout(user_table.T, item_table.T)
    u_pad, i_pad = _sc_gather(
        uids.reshape(NUM_WORKERS, ROWS_PER_WORKER),
        iids.reshape(NUM_WORKERS, ROWS_PER_WORKER),
        utab, itab)
    return _mlp(u_pad, i_pad, uids, iids, W1, b1, W2, b2, W3, b3)


# speedup vs baseline: 1.0013x; 1.0013x over previous
import functools

import jax
import jax.numpy as jnp
from jax import lax
from jax.experimental import pallas as pl
from jax.experimental.pallas import tpu as pltpu
from jax.experimental.pallas import tpu_sc as plsc

BATCH = 16384
EMBED_DIM = 32
NUM_ROWS = 1000000
PACK = 4
PAD_DIM = PACK * EMBED_DIM
PACKED_ROWS = 253952
NUM_CORES = 2
NUM_SUBCORES = 16
NUM_WORKERS = NUM_CORES * NUM_SUBCORES
ROWS_PER_WORKER = BATCH // NUM_WORKERS
CHUNK = 128
NUM_CHUNKS = ROWS_PER_WORKER // CHUNK
LANES = 16


RELAY_P = 4096
RELAY_GRID = PACKED_ROWS // RELAY_P


def _relayout_body(u0, u1, u2, u3, i0, i1, i2, i3, ou_ref, oi_ref):
    eye = jnp.eye(EMBED_DIM, dtype=jnp.float32)
    tr = lambda x: lax.dot_general(
        x[...], eye, (((0,), (0,)), ((), ())),
        preferred_element_type=jnp.float32)
    for k, (xu, xi) in enumerate(zip((u0, u1, u2, u3), (i0, i1, i2, i3))):
        sl = pl.ds(k * EMBED_DIM, EMBED_DIM)
        ou_ref[:, sl] = tr(xu)
        oi_ref[:, sl] = tr(xi)


def _relayout(utabT, itabT):
    spec = [
        pl.BlockSpec(
            (EMBED_DIM, RELAY_P),
            functools.partial(
                lambda k, i: (0, jnp.minimum(i + k * RELAY_GRID,
                                             NUM_ROWS // RELAY_P)), k))
        for k in range(PACK)
    ]
    return pl.pallas_call(
        _relayout_body,
        grid=(RELAY_GRID,),
        in_specs=spec + spec,
        out_specs=[pl.BlockSpec((RELAY_P, PAD_DIM), lambda i: (i, 0))] * 2,
        out_shape=[jax.ShapeDtypeStruct((PACKED_ROWS, PAD_DIM), jnp.float32)] * 2,
    )(utabT, utabT, utabT, utabT, itabT, itabT, itabT, itabT)


_sc_mesh = plsc.VectorSubcoreMesh(core_axis_name="c", subcore_axis_name="s")


@functools.partial(
    pl.kernel,
    mesh=_sc_mesh,
    out_type=[
        jax.ShapeDtypeStruct((BATCH, PAD_DIM), jnp.float32),
        jax.ShapeDtypeStruct((BATCH, PAD_DIM), jnp.float32),
    ],
    scratch_types=[
        pltpu.VMEM((ROWS_PER_WORKER,), jnp.int32),
        pltpu.VMEM((ROWS_PER_WORKER,), jnp.int32),
        pltpu.VMEM((2, CHUNK, PAD_DIM), jnp.float32),
        pltpu.VMEM((2, CHUNK, PAD_DIM), jnp.float32),
        pltpu.SemaphoreType.DMA,
        pltpu.SemaphoreType.DMA,
    ],
    compiler_params=pltpu.CompilerParams(use_tc_tiling_on_sc=False),
)
def _sc_gather(uids_hbm, iids_hbm, utab_hbm, itab_hbm, uout_hbm, iout_hbm,
               uidx_v, iidx_v, upad_v, ipad_v, sem_g, sem_o):
    wid = lax.axis_index("s") * NUM_CORES + lax.axis_index("c")
    base = wid * ROWS_PER_WORKER
    pltpu.sync_copy(uids_hbm.at[wid], uidx_v)
    pltpu.sync_copy(iids_hbm.at[wid], iidx_v)
    for g in range(ROWS_PER_WORKER // LANES):
        sl = pl.ds(g * LANES, LANES)
        uidx_v[sl] = lax.rem(uidx_v[sl], PACKED_ROWS)
        iidx_v[sl] = lax.rem(iidx_v[sl], PACKED_ROWS)

    def fire(j):
        buf = j % 2
        return (
            pltpu.async_copy(
                utab_hbm.at[uidx_v.at[pl.ds(j * CHUNK, CHUNK)]],
                upad_v.at[buf], sem_g),
            pltpu.async_copy(
                itab_hbm.at[iidx_v.at[pl.ds(j * CHUNK, CHUNK)]],
                ipad_v.at[buf], sem_g),
        )

    def flush(j):
        buf = j % 2
        dst = pl.ds(base + j * CHUNK, CHUNK)
        return (
            pltpu.async_copy(upad_v.at[buf], uout_hbm.at[dst], sem_o),
            pltpu.async_copy(ipad_v.at[buf], iout_hbm.at[dst], sem_o),
        )

    gathers = fire(0)
    outs = []
    for j in range(NUM_CHUNKS):
        for c in gathers:
            c.wait()
        outs.append(flush(j))
        if j + 1 < NUM_CHUNKS:
            if j >= 1:
                for c in outs[j - 1]:
                    c.wait()
            gathers = fire(j + 1)
    for pair in outs[-2:]:
        for c in pair:
            c.wait()


MLP_BLOCK = 2048


def _mlp_body(u_ref, i_ref, uid_ref, iid_ref, w1u_ref, w1i_ref, b1_ref,
              w2_ref, b2_ref, w3_ref, b3_ref, o_ref):
    lane_group = lax.broadcasted_iota(jnp.int32, (MLP_BLOCK, PAD_DIM), 1) // EMBED_DIM
    u_sel = jnp.where(lane_group == uid_ref[...] // PACKED_ROWS, u_ref[...], 0.0)
    i_sel = jnp.where(lane_group == iid_ref[...] // PACKED_ROWS, i_ref[...], 0.0)
    h = jnp.dot(u_sel, w1u_ref[...], preferred_element_type=jnp.float32)
    h = h + jnp.dot(i_sel, w1i_ref[...], preferred_element_type=jnp.float32)
    h = jnp.maximum(h + b1_ref[...], 0.0)
    h = jnp.dot(h, w2_ref[...], preferred_element_type=jnp.float32) + b2_ref[...]
    h = jnp.maximum(h, 0.0)
    o_ref[...] = (
        jnp.dot(h, w3_ref[...], preferred_element_type=jnp.float32) + b3_ref[...]
    )


def _mlp(u_pad, i_pad, uids, iids, W1, b1, W2, b2, W3, b3):
    w1u = jnp.concatenate([W1[:EMBED_DIM]] * PACK, axis=0)
    w1i = jnp.concatenate([W1[EMBED_DIM:]] * PACK, axis=0)
    grid = (BATCH // MLP_BLOCK,)
    full = lambda shape: pl.BlockSpec(shape, lambda i: (0, 0))
    out = pl.pallas_call(
        _mlp_body,
        grid=grid,
        in_specs=[
            pl.BlockSpec((MLP_BLOCK, PAD_DIM), lambda i: (i, 0)),
            pl.BlockSpec((MLP_BLOCK, PAD_DIM), lambda i: (i, 0)),
            pl.BlockSpec((MLP_BLOCK, 1), lambda i: (i, 0)),
            pl.BlockSpec((MLP_BLOCK, 1), lambda i: (i, 0)),
            full((PAD_DIM, 64)),
            full((PAD_DIM, 64)),
            full((1, 64)),
            full((64, 32)),
            full((1, 32)),
            full((32, 1)),
            full((1, 1)),
        ],
        out_specs=pl.BlockSpec((MLP_BLOCK, 1), lambda i: (i, 0)),
        out_shape=jax.ShapeDtypeStruct((BATCH, 1), jnp.float32),
    )(u_pad, i_pad, uids.reshape(BATCH, 1), iids.reshape(BATCH, 1), w1u, w1i,
      b1.reshape(1, 64), W2, b2.reshape(1, 32), W3, b3.reshape(1, 1))
    return out[:, 0]


def kernel(user_ids, item_ids, user_table, item_table, W1, b1, W2, b2, W3, b3):
    uids = user_ids.astype(jnp.int32)
    iids = item_ids.astype(jnp.int32)
    utab, itab = _relayout(user_table.T, item_table.T)
    u_pad, i_pad = _sc_gather(
        uids.reshape(NUM_WORKERS, ROWS_PER_WORKER),
        iids.reshape(NUM_WORKERS, ROWS_PER_WORKER),
        utab, itab)
    return _mlp(u_pad, i_pad, uids, iids, W1, b1, W2, b2, W3, b3)


# speedup vs baseline: 1.0089x; 1.0076x over previous
import functools

import jax
import jax.numpy as jnp
from jax import lax
from jax.experimental import pallas as pl
from jax.experimental.pallas import tpu as pltpu
from jax.experimental.pallas import tpu_sc as plsc

BATCH = 16384
EMBED_DIM = 32
NUM_ROWS = 1000000
PACK = 4
PAD_DIM = PACK * EMBED_DIM
PACKED_ROWS = 253952
NUM_CORES = 2
NUM_SUBCORES = 16
NUM_WORKERS = NUM_CORES * NUM_SUBCORES
ROWS_PER_WORKER = BATCH // NUM_WORKERS
CHUNK = 128
NUM_CHUNKS = ROWS_PER_WORKER // CHUNK
LANES = 16


RELAY_P = 8192
RELAY_GRID = PACKED_ROWS // RELAY_P


def _relayout_body(u0, u1, u2, u3, i0, i1, i2, i3, ou_ref, oi_ref):
    eye = jnp.eye(EMBED_DIM, dtype=jnp.float32)
    tr = lambda x: lax.dot_general(
        x[...], eye, (((0,), (0,)), ((), ())),
        preferred_element_type=jnp.float32)
    for k, (xu, xi) in enumerate(zip((u0, u1, u2, u3), (i0, i1, i2, i3))):
        sl = pl.ds(k * EMBED_DIM, EMBED_DIM)
        ou_ref[:, sl] = tr(xu)
        oi_ref[:, sl] = tr(xi)


def _relayout(utabT, itabT):
    spec = [
        pl.BlockSpec(
            (EMBED_DIM, RELAY_P),
            functools.partial(
                lambda k, i: (0, jnp.minimum(i + k * RELAY_GRID,
                                             NUM_ROWS // RELAY_P)), k))
        for k in range(PACK)
    ]
    return pl.pallas_call(
        _relayout_body,
        grid=(RELAY_GRID,),
        in_specs=spec + spec,
        out_specs=[pl.BlockSpec((RELAY_P, PAD_DIM), lambda i: (i, 0))] * 2,
        out_shape=[jax.ShapeDtypeStruct((PACKED_ROWS, PAD_DIM), jnp.float32)] * 2,
    )(utabT, utabT, utabT, utabT, itabT, itabT, itabT, itabT)


_sc_mesh = plsc.VectorSubcoreMesh(core_axis_name="c", subcore_axis_name="s")


@functools.partial(
    pl.kernel,
    mesh=_sc_mesh,
    out_type=[
        jax.ShapeDtypeStruct((BATCH, PAD_DIM), jnp.float32),
        jax.ShapeDtypeStruct((BATCH, PAD_DIM), jnp.float32),
    ],
    scratch_types=[
        pltpu.VMEM((ROWS_PER_WORKER,), jnp.int32),
        pltpu.VMEM((ROWS_PER_WORKER,), jnp.int32),
        pltpu.VMEM((2, CHUNK, PAD_DIM), jnp.float32),
        pltpu.VMEM((2, CHUNK, PAD_DIM), jnp.float32),
        pltpu.SemaphoreType.DMA,
        pltpu.SemaphoreType.DMA,
    ],
    compiler_params=pltpu.CompilerParams(use_tc_tiling_on_sc=False),
)
def _sc_gather(uids_hbm, iids_hbm, utab_hbm, itab_hbm, uout_hbm, iout_hbm,
               uidx_v, iidx_v, upad_v, ipad_v, sem_g, sem_o):
    wid = lax.axis_index("s") * NUM_CORES + lax.axis_index("c")
    base = wid * ROWS_PER_WORKER
    pltpu.sync_copy(uids_hbm.at[wid], uidx_v)
    pltpu.sync_copy(iids_hbm.at[wid], iidx_v)
    for g in range(ROWS_PER_WORKER // LANES):
        sl = pl.ds(g * LANES, LANES)
        uidx_v[sl] = lax.rem(uidx_v[sl], PACKED_ROWS)
        iidx_v[sl] = lax.rem(iidx_v[sl], PACKED_ROWS)

    def fire(j):
        buf = j % 2
        return (
            pltpu.async_copy(
                utab_hbm.at[uidx_v.at[pl.ds(j * CHUNK, CHUNK)]],
                upad_v.at[buf], sem_g),
            pltpu.async_copy(
                itab_hbm.at[iidx_v.at[pl.ds(j * CHUNK, CHUNK)]],
                ipad_v.at[buf], sem_g),
        )

    def flush(j):
        buf = j % 2
        dst = pl.ds(base + j * CHUNK, CHUNK)
        return (
            pltpu.async_copy(upad_v.at[buf], uout_hbm.at[dst], sem_o),
            pltpu.async_copy(ipad_v.at[buf], iout_hbm.at[dst], sem_o),
        )

    gathers = fire(0)
    outs = []
    for j in range(NUM_CHUNKS):
        for c in gathers:
            c.wait()
        outs.append(flush(j))
        if j + 1 < NUM_CHUNKS:
            if j >= 1:
                for c in outs[j - 1]:
                    c.wait()
            gathers = fire(j + 1)
    for pair in outs[-2:]:
        for c in pair:
            c.wait()


MLP_BLOCK = 2048


def _mlp_body(u_ref, i_ref, uid_ref, iid_ref, w1u_ref, w1i_ref, b1_ref,
              w2_ref, b2_ref, w3_ref, b3_ref, o_ref):
    lane_group = lax.broadcasted_iota(jnp.int32, (MLP_BLOCK, PAD_DIM), 1) // EMBED_DIM
    u_sel = jnp.where(lane_group == uid_ref[...] // PACKED_ROWS, u_ref[...], 0.0)
    i_sel = jnp.where(lane_group == iid_ref[...] // PACKED_ROWS, i_ref[...], 0.0)
    h = jnp.dot(u_sel, w1u_ref[...], preferred_element_type=jnp.float32)
    h = h + jnp.dot(i_sel, w1i_ref[...], preferred_element_type=jnp.float32)
    h = jnp.maximum(h + b1_ref[...], 0.0)
    h = jnp.dot(h, w2_ref[...], preferred_element_type=jnp.float32) + b2_ref[...]
    h = jnp.maximum(h, 0.0)
    o_ref[...] = (
        jnp.dot(h, w3_ref[...], preferred_element_type=jnp.float32) + b3_ref[...]
    )


def _mlp(u_pad, i_pad, uids, iids, W1, b1, W2, b2, W3, b3):
    w1u = jnp.concatenate([W1[:EMBED_DIM]] * PACK, axis=0)
    w1i = jnp.concatenate([W1[EMBED_DIM:]] * PACK, axis=0)
    grid = (BATCH // MLP_BLOCK,)
    full = lambda shape: pl.BlockSpec(shape, lambda i: (0, 0))
    out = pl.pallas_call(
        _mlp_body,
        grid=grid,
        in_specs=[
            pl.BlockSpec((MLP_BLOCK, PAD_DIM), lambda i: (i, 0)),
            pl.BlockSpec((MLP_BLOCK, PAD_DIM), lambda i: (i, 0)),
            pl.BlockSpec((MLP_BLOCK, 1), lambda i: (i, 0)),
            pl.BlockSpec((MLP_BLOCK, 1), lambda i: (i, 0)),
            full((PAD_DIM, 64)),
            full((PAD_DIM, 64)),
            full((1, 64)),
            full((64, 32)),
            full((1, 32)),
            full((32, 1)),
            full((1, 1)),
        ],
        out_specs=pl.BlockSpec((MLP_BLOCK, 1), lambda i: (i, 0)),
        out_shape=jax.ShapeDtypeStruct((BATCH, 1), jnp.float32),
    )(u_pad, i_pad, uids.reshape(BATCH, 1), iids.reshape(BATCH, 1), w1u, w1i,
      b1.reshape(1, 64), W2, b2.reshape(1, 32), W3, b3.reshape(1, 1))
    return out[:, 0]


def kernel(user_ids, item_ids, user_table, item_table, W1, b1, W2, b2, W3, b3):
    uids = user_ids.astype(jnp.int32)
    iids = item_ids.astype(jnp.int32)
    utab, itab = _relayout(user_table.T, item_table.T)
    u_pad, i_pad = _sc_gather(
        uids.reshape(NUM_WORKERS, ROWS_PER_WORKER),
        iids.reshape(NUM_WORKERS, ROWS_PER_WORKER),
        utab, itab)
    return _mlp(u_pad, i_pad, uids, iids, W1, b1, W2, b2, W3, b3)
